# baseline (speedup 1.0000x reference)
import jax
import jax.numpy as jnp
from jax.experimental import pallas as pl
from jax.experimental.pallas import tpu as pltpu

_ROWS, _COLS = 50176, 768
_CHUNK = 12544
_N_CHUNKS = _ROWS // _CHUNK
_NDMA = 4
_SUB = _CHUNK // _NDMA


def _phased_body(x_hbm, o_hbm, buf, sems):
    for c in range(_N_CHUNKS):
        base = c * _CHUNK
        in_cps = [
            pltpu.make_async_copy(
                x_hbm.at[pl.ds(base + j * _SUB, _SUB)],
                buf.at[pl.ds(j * _SUB, _SUB)],
                sems.at[j],
            )
            for j in range(_NDMA)
        ]
        for cp in in_cps:
            cp.start()
        for cp in in_cps:
            cp.wait()
        out_cps = [
            pltpu.make_async_copy(
                buf.at[pl.ds(j * _SUB, _SUB)],
                o_hbm.at[pl.ds(base + j * _SUB, _SUB)],
                sems.at[j],
            )
            for j in range(_NDMA)
        ]
        for cp in out_cps:
            cp.start()
        for cp in out_cps:
            cp.wait()


def kernel(x, center):
    del center
    flat = x.transpose(2, 3, 0, 1).reshape(_ROWS, _COLS)
    yt = pl.pallas_call(
        _phased_body,
        in_specs=[pl.BlockSpec(memory_space=pltpu.MemorySpace.HBM)],
        out_specs=pl.BlockSpec(memory_space=pltpu.MemorySpace.HBM),
        out_shape=jax.ShapeDtypeStruct((_ROWS, _COLS), x.dtype),
        scratch_shapes=[
            pltpu.VMEM((_CHUNK, _COLS), jnp.float32),
            pltpu.SemaphoreType.DMA((_NDMA,)),
        ],
    )(flat)
    return yt.reshape(14, 14, 256, 768).transpose(2, 3, 0, 1)


# speedup vs baseline: 1.0005x; 1.0005x over previous
import jax
import jax.numpy as jnp
from jax.experimental import pallas as pl
from jax.experimental.pallas import tpu as pltpu

_ROWS, _COLS = 50176, 768
_CHUNK = 12544
_N_CHUNKS = _ROWS // _CHUNK
_NDMA = 8
_SUB = _CHUNK // _NDMA


def _phased_body(x_hbm, o_hbm, buf, sems):
    for c in range(_N_CHUNKS):
        base = c * _CHUNK
        in_cps = [
            pltpu.make_async_copy(
                x_hbm.at[pl.ds(base + j * _SUB, _SUB)],
                buf.at[pl.ds(j * _SUB, _SUB)],
                sems.at[j],
            )
            for j in range(_NDMA)
        ]
        for cp in in_cps:
            cp.start()
        for cp in in_cps:
            cp.wait()
        out_cps = [
            pltpu.make_async_copy(
                buf.at[pl.ds(j * _SUB, _SUB)],
                o_hbm.at[pl.ds(base + j * _SUB, _SUB)],
                sems.at[j],
            )
            for j in range(_NDMA)
        ]
        for cp in out_cps:
            cp.start()
        for cp in out_cps:
            cp.wait()


def kernel(x, center):
    del center
    flat = x.transpose(2, 3, 0, 1).reshape(_ROWS, _COLS)
    yt = pl.pallas_call(
        _phased_body,
        in_specs=[pl.BlockSpec(memory_space=pltpu.MemorySpace.HBM)],
        out_specs=pl.BlockSpec(memory_space=pltpu.MemorySpace.HBM),
        out_shape=jax.ShapeDtypeStruct((_ROWS, _COLS), x.dtype),
        scratch_shapes=[
            pltpu.VMEM((_CHUNK, _COLS), jnp.float32),
            pltpu.SemaphoreType.DMA((_NDMA,)),
        ],
    )(flat)
    return yt.reshape(14, 14, 256, 768).transpose(2, 3, 0, 1)


# speedup vs baseline: 1.0393x; 1.0387x over previous
import jax
import jax.numpy as jnp
from jax.experimental import pallas as pl
from jax.experimental.pallas import tpu as pltpu

_ROWS, _COLS = 50176, 768
_BLK = 3584


def _copy_body(x_ref, o_ref):
    o_ref[...] = x_ref[...]


def kernel(x, center):
    del center
    flat = x.transpose(2, 3, 0, 1).reshape(_ROWS, _COLS)
    yt = pl.pallas_call(
        _copy_body,
        grid=(_ROWS // _BLK,),
        in_specs=[pl.BlockSpec((_BLK, _COLS), lambda i: (i, 0))],
        out_specs=pl.BlockSpec((_BLK, _COLS), lambda i: (i, 0)),
        out_shape=jax.ShapeDtypeStruct((_ROWS, _COLS), x.dtype),
        compiler_params=pltpu.CompilerParams(
            dimension_semantics=("arbitrary",),
        ),
    )(flat)
    return yt.reshape(14, 14, 256, 768).transpose(2, 3, 0, 1)
